# Initial kernel scaffold; baseline (speedup 1.0000x reference)
import jax
import jax.numpy as jnp
from jax.experimental import pallas as pl


def kernel(x, W1, b1, W2, b2, W3, b3, Wf1, bf1, Wf2, bf2, Wf3, bf3):
    raise NotImplementedError("write your pallas kernel here")


# speedup vs baseline: 7.4674x; 7.4674x over previous
import functools

import jax
import jax.numpy as jnp
from jax import lax
from jax.experimental import pallas as pl
from jax.experimental.pallas import tpu as pltpu

_TOPK = 32


def _stage1(x_ref, xt_ref, w1_ref, b1_ref, w2_ref, b2_ref, w3_ref, b3_ref,
            g_ref, nd_s, m_s, *, n_points, rows):
    t = pl.program_id(1)
    xb = x_ref[0]
    xtt = xt_ref[0]

    f32 = jnp.float32
    z = jnp.maximum(
        jax.lax.dot_general(w1_ref[...], xb, (((1,), (0,)), ((), ())),
                            preferred_element_type=f32) + b1_ref[...], 0.0)
    q = jax.lax.dot_general(w2_ref[...], z, (((1,), (0,)), ((), ())),
                            preferred_element_type=f32) + b2_ref[...]

    xxc = jnp.sum(xb * xb, axis=0, keepdims=True)
    xxr = jnp.sum(xtt * xtt, axis=1, keepdims=True)
    nd_s[...] = (2.0 * jax.lax.dot_general(
        xtt, xb, (((1,), (0,)), ((), ())), preferred_element_type=f32)
        - xxr - xxc)
    m_s[...] = jnp.full((rows, 128), -jnp.inf, f32)

    iota = lax.broadcasted_iota(jnp.int32, (rows, n_points), 1)

    def body(_, carry):
        nd = nd_s[...]
        rowmax = jnp.max(nd, axis=1, keepdims=True)
        selcol = jnp.min(jnp.where(nd == rowmax, iota, n_points),
                         axis=1, keepdims=True)
        onehot = (iota == selcol)
        contrib = jax.lax.dot_general(
            onehot.astype(f32), q, (((1,), (1,)), ((), ())),
            preferred_element_type=f32)
        m_s[...] = jnp.maximum(m_s[...], contrib)
        nd_s[...] = jnp.where(onehot, -jnp.inf, nd)
        return carry

    lax.fori_loop(0, _TOPK, body, 0)

    mr = jnp.maximum(m_s[...], 0.0)
    h3 = jnp.maximum(
        jax.lax.dot_general(mr, w3_ref[...], (((1,), (1,)), ((), ())),
                            preferred_element_type=f32) + b3_ref[...], 0.0)
    part = jnp.max(h3, axis=0, keepdims=True)

    @pl.when(t == 0)
    def _():
        g_ref[0] = part

    @pl.when(t > 0)
    def _():
        g_ref[0] = jnp.maximum(g_ref[0], part)


def _stage2(g_ref, wf1_ref, bf1_ref, wf2_ref, bf2_ref, wf3_ref, bf3_ref,
            x_ref, o_ref, *, c_dim):
    f32 = jnp.float32
    gb = g_ref[0]
    h1 = jnp.maximum(
        jax.lax.dot_general(gb, wf1_ref[...], (((1,), (1,)), ((), ())),
                            preferred_element_type=f32) + bf1_ref[...], 0.0)
    h2 = jnp.maximum(
        jax.lax.dot_general(h1, wf2_ref[...], (((1,), (1,)), ((), ())),
                            preferred_element_type=f32) + bf2_ref[...], 0.0)
    t9 = jax.lax.dot_general(h2, wf3_ref[...], (((1,), (1,)), ((), ())),
                             preferred_element_type=f32) + bf3_ref[...]
    i9 = lax.broadcasted_iota(jnp.int32, (1, c_dim * c_dim), 1)
    t9 = t9 + jnp.where(i9 % (c_dim + 1) == 0, 1.0, 0.0)

    xb = x_ref[0]
    rows = []
    for d in range(c_dim):
        acc = xb[0:1, :] * t9[0:1, d:d + 1]
        for c in range(1, c_dim):
            acc = acc + xb[c:c + 1, :] * t9[0:1, c_dim * c + d:c_dim * c + d + 1]
        rows.append(acc)
    o_ref[0] = jnp.concatenate(rows, axis=0)


def kernel(x, W1, b1, W2, b2, W3, b3, Wf1, bf1, Wf2, bf2, Wf3, bf3):
    B, C, N = x.shape
    R = min(512, N)
    T = N // R
    xt = jnp.transpose(x, (0, 2, 1))
    b1c = b1.reshape(-1, 1)
    b2c = b2.reshape(-1, 1)
    b3r = b3.reshape(1, -1)

    g = pl.pallas_call(
        functools.partial(_stage1, n_points=N, rows=R),
        grid=(B, T),
        in_specs=[
            pl.BlockSpec((1, C, N), lambda b, t: (b, 0, 0)),
            pl.BlockSpec((1, R, C), lambda b, t: (b, t, 0)),
            pl.BlockSpec(W1.shape, lambda b, t: (0, 0)),
            pl.BlockSpec(b1c.shape, lambda b, t: (0, 0)),
            pl.BlockSpec(W2.shape, lambda b, t: (0, 0)),
            pl.BlockSpec(b2c.shape, lambda b, t: (0, 0)),
            pl.BlockSpec(W3.shape, lambda b, t: (0, 0)),
            pl.BlockSpec(b3r.shape, lambda b, t: (0, 0)),
        ],
        out_specs=pl.BlockSpec((1, 1, 1024), lambda b, t: (b, 0, 0)),
        out_shape=jax.ShapeDtypeStruct((B, 1, 1024), jnp.float32),
        scratch_shapes=[
            pltpu.VMEM((R, N), jnp.float32),
            pltpu.VMEM((R, 128), jnp.float32),
        ],
    )(x, xt, W1, b1c, W2, b2c, W3, b3r)

    bf1r = bf1.reshape(1, -1)
    bf2r = bf2.reshape(1, -1)
    bf3r = bf3.reshape(1, -1)
    out = pl.pallas_call(
        functools.partial(_stage2, c_dim=C),
        grid=(B,),
        in_specs=[
            pl.BlockSpec((1, 1, 1024), lambda b: (b, 0, 0)),
            pl.BlockSpec(Wf1.shape, lambda b: (0, 0)),
            pl.BlockSpec(bf1r.shape, lambda b: (0, 0)),
            pl.BlockSpec(Wf2.shape, lambda b: (0, 0)),
            pl.BlockSpec(bf2r.shape, lambda b: (0, 0)),
            pl.BlockSpec(Wf3.shape, lambda b: (0, 0)),
            pl.BlockSpec(bf3r.shape, lambda b: (0, 0)),
            pl.BlockSpec((1, C, N), lambda b: (b, 0, 0)),
        ],
        out_specs=pl.BlockSpec((1, C, N), lambda b: (b, 0, 0)),
        out_shape=jax.ShapeDtypeStruct((B, C, N), jnp.float32),
    )(g, Wf1, bf1r, Wf2, bf2r, Wf3, bf3r, x)
    return out


# speedup vs baseline: 7.7719x; 1.0408x over previous
import functools

import jax
import jax.numpy as jnp
from jax import lax
from jax.experimental import pallas as pl
from jax.experimental.pallas import tpu as pltpu
from jax.experimental.pallas import tpu_sc as plsc

_TOPK = 32
_NEG = -3.0e38


def _stage_a(x_ref, xt_ref, w1_ref, b1_ref, w2_ref, b2_ref, qt_ref, xx_ref):
    f32 = jnp.float32
    xb = x_ref[0]
    xtt = xt_ref[0]
    zt = jnp.maximum(
        jax.lax.dot_general(xtt, w1_ref[...], (((1,), (1,)), ((), ())),
                            preferred_element_type=f32) + b1_ref[...], 0.0)
    qt = jax.lax.dot_general(zt, w2_ref[...], (((1,), (1,)), ((), ())),
                             preferred_element_type=f32) + b2_ref[...]
    qt_ref[0] = qt
    xx_ref[0] = jnp.sum(xb * xb, axis=0, keepdims=True)


def _bmax(v, iota16):
    for k in (1, 2, 4, 8):
        v = jnp.maximum(v, v.at[iota16 ^ k].get(mode="promise_in_bounds"))
    return v


def _sc_body(xc_hbm, xx_hbm, qt_hbm, m_hbm,
             xv0, xv1, xv2, xxv, ndv, cmaxv, idxv, qbuf, mbuf, sem,
             *, n_points, batch, rows_per_worker):
    f32 = jnp.float32
    nc = 2
    wid = lax.axis_index("s") * nc + lax.axis_index("c")
    wpb = n_points // rows_per_worker
    b = wid // wpb
    r0 = (wid % wpb) * rows_per_worker

    pltpu.sync_copy(xc_hbm.at[3 * b + 0], xv0)
    pltpu.sync_copy(xc_hbm.at[3 * b + 1], xv1)
    pltpu.sync_copy(xc_hbm.at[3 * b + 2], xv2)
    pltpu.sync_copy(xx_hbm.at[b], xxv)

    iota16 = lax.iota(jnp.int32, 16)
    n_groups = n_points // 256

    def row_body(i, _):
        n = r0 + i
        nsplat = jnp.full((16,), 0, jnp.int32) + n
        xn0 = plsc.load_gather(xv0, [nsplat])
        xn1 = plsc.load_gather(xv1, [nsplat])
        xn2 = plsc.load_gather(xv2, [nsplat])

        def group_body(g, l2reg):
            gb = g * 256
            cmreg = jnp.full((16,), _NEG, f32)
            for jj in range(16):
                bidx = gb + (jj * 16 + iota16)
                v = (2.0 * (xn0 * plsc.load_gather(xv0, [bidx])
                            + xn1 * plsc.load_gather(xv1, [bidx])
                            + xn2 * plsc.load_gather(xv2, [bidx]))
                     - plsc.load_gather(xxv, [bidx]))
                plsc.store_scatter(ndv, [bidx], v)
                cmreg = jnp.where(iota16 == jj, _bmax(v, iota16), cmreg)
            plsc.store_scatter(cmaxv, [g * 16 + iota16], cmreg)
            return jnp.where(iota16 == g, _bmax(cmreg, iota16), l2reg)

        l2reg = lax.fori_loop(0, n_groups, group_body,
                              jnp.full((16,), _NEG, f32))

        def round_body(k, carry):
            l2reg, idx_a, idx_b = carry
            s = _bmax(l2reg, iota16)
            ggv = _bmax(jnp.where(l2reg == s, iota16, -1), iota16)
            cm16 = plsc.load_gather(cmaxv, [ggv * 16 + iota16])
            ccv = _bmax(jnp.where(cm16 == s, iota16, -1), iota16)
            cv = ggv * 16 + ccv
            didx = cv * 16 + iota16
            dchunk = plsc.load_gather(ndv, [didx])
            llv = _bmax(jnp.where(dchunk == s, iota16, -1), iota16)
            colv = cv * 16 + llv + b * n_points
            idx_a = jnp.where(iota16 == k, colv, idx_a)
            idx_b = jnp.where(iota16 == k - 16, colv, idx_b)
            dchunk = jnp.where(iota16 == llv, _NEG, dchunk)
            plsc.store_scatter(ndv, [didx], dchunk)
            ncm = _bmax(dchunk, iota16)
            cm16 = jnp.where(iota16 == ccv, ncm, cm16)
            plsc.store_scatter(cmaxv, [ggv * 16 + iota16], cm16)
            nl2 = _bmax(cm16, iota16)
            l2reg = jnp.where(iota16 == ggv, nl2, l2reg)
            return l2reg, idx_a, idx_b

        zi = jnp.zeros((16,), jnp.int32)
        _, idx_a, idx_b = lax.fori_loop(0, _TOPK, round_body,
                                        (l2reg, zi, zi))
        idxv[pl.ds(0, 16)] = idx_a
        idxv[pl.ds(16, 16)] = idx_b

        pltpu.async_copy(qt_hbm.at[idxv], qbuf, sem).wait()

        rowsplat = jnp.full((16,), 0, jnp.int32) + (i % 32)
        for cc8 in range(8):
            cols = cc8 * 16 + iota16
            acc = jnp.full((16,), _NEG, f32)
            for r in range(_TOPK):
                rs = jnp.full((16,), r, jnp.int32)
                acc = jnp.maximum(acc, plsc.load_gather(qbuf, [rs, cols]))
            plsc.store_scatter(mbuf, [rowsplat, cols], acc)

        @pl.when(i % 32 == 31)
        def _():
            pltpu.sync_copy(
                mbuf, m_hbm.at[pl.ds(b * n_points + r0 + (i // 32) * 32, 32)])
        return 0

    lax.fori_loop(0, rows_per_worker, row_body, 0)


def _stage_c(m_ref, w3_ref, b3_ref, g_ref):
    f32 = jnp.float32
    t = pl.program_id(1)
    mr = jnp.maximum(m_ref[...], 0.0)
    h3 = jnp.maximum(
        jax.lax.dot_general(mr, w3_ref[...], (((1,), (1,)), ((), ())),
                            preferred_element_type=f32) + b3_ref[...], 0.0)
    part = jnp.max(h3, axis=0, keepdims=True)

    @pl.when(t == 0)
    def _():
        g_ref[0] = part

    @pl.when(t > 0)
    def _():
        g_ref[0] = jnp.maximum(g_ref[0], part)


def _stage_d(g_ref, wf1_ref, bf1_ref, wf2_ref, bf2_ref, wf3_ref, bf3_ref,
             x_ref, o_ref, *, c_dim):
    f32 = jnp.float32
    gb = g_ref[0]
    h1 = jnp.maximum(
        jax.lax.dot_general(gb, wf1_ref[...], (((1,), (1,)), ((), ())),
                            preferred_element_type=f32) + bf1_ref[...], 0.0)
    h2 = jnp.maximum(
        jax.lax.dot_general(h1, wf2_ref[...], (((1,), (1,)), ((), ())),
                            preferred_element_type=f32) + bf2_ref[...], 0.0)
    t9 = jax.lax.dot_general(h2, wf3_ref[...], (((1,), (1,)), ((), ())),
                             preferred_element_type=f32) + bf3_ref[...]
    i9 = lax.broadcasted_iota(jnp.int32, (1, c_dim * c_dim), 1)
    t9 = t9 + jnp.where(i9 % (c_dim + 1) == 0, 1.0, 0.0)

    xb = x_ref[0]
    rows = []
    for d in range(c_dim):
        acc = xb[0:1, :] * t9[0:1, d:d + 1]
        for c in range(1, c_dim):
            acc = acc + xb[c:c + 1, :] * t9[0:1, c_dim * c + d:c_dim * c + d + 1]
        rows.append(acc)
    o_ref[0] = jnp.concatenate(rows, axis=0)


def kernel(x, W1, b1, W2, b2, W3, b3, Wf1, bf1, Wf2, bf2, Wf3, bf3):
    B, C, N = x.shape
    f32 = jnp.float32
    xt = jnp.transpose(x, (0, 2, 1))
    xc = x.reshape(B * C, N)
    b1r = b1.reshape(1, -1)
    b2r = b2.reshape(1, -1)
    b3r = b3.reshape(1, -1)

    qt, xx = pl.pallas_call(
        _stage_a,
        grid=(B,),
        in_specs=[
            pl.BlockSpec((1, C, N), lambda b: (b, 0, 0)),
            pl.BlockSpec((1, N, C), lambda b: (b, 0, 0)),
            pl.BlockSpec(W1.shape, lambda b: (0, 0)),
            pl.BlockSpec(b1r.shape, lambda b: (0, 0)),
            pl.BlockSpec(W2.shape, lambda b: (0, 0)),
            pl.BlockSpec(b2r.shape, lambda b: (0, 0)),
        ],
        out_specs=[
            pl.BlockSpec((1, N, 128), lambda b: (b, 0, 0)),
            pl.BlockSpec((1, 1, N), lambda b: (b, 0, 0)),
        ],
        out_shape=[
            jax.ShapeDtypeStruct((B, N, 128), f32),
            jax.ShapeDtypeStruct((B, 1, N), f32),
        ],
    )(x, xt, W1, b1r, W2, b2r)
    qtf = qt.reshape(B * N, 128)
    xx2 = xx.reshape(B, N)

    n_workers = 32
    rpw = (B * N) // n_workers
    mesh = plsc.VectorSubcoreMesh(core_axis_name="c", subcore_axis_name="s")
    sc = functools.partial(
        pl.kernel,
        mesh=mesh,
        compiler_params=pltpu.CompilerParams(needs_layout_passes=False),
        out_type=jax.ShapeDtypeStruct((B * N, 128), f32),
        scratch_types=[
            pltpu.VMEM((N,), f32),
            pltpu.VMEM((N,), f32),
            pltpu.VMEM((N,), f32),
            pltpu.VMEM((N,), f32),
            pltpu.VMEM((N,), f32),
            pltpu.VMEM((N // 16,), f32),
            pltpu.VMEM((_TOPK,), jnp.int32),
            pltpu.VMEM((_TOPK, 128), f32),
            pltpu.VMEM((32, 128), f32),
            pltpu.SemaphoreType.DMA,
        ],
    )(functools.partial(_sc_body, n_points=N, batch=B, rows_per_worker=rpw))
    m = sc(xc, xx2, qtf)

    R2 = 1024
    T2 = (B * N) // R2 // B
    g = pl.pallas_call(
        _stage_c,
        grid=(B, T2),
        in_specs=[
            pl.BlockSpec((R2, 128), lambda b, t: (b * T2 + t, 0)),
            pl.BlockSpec(W3.shape, lambda b, t: (0, 0)),
            pl.BlockSpec(b3r.shape, lambda b, t: (0, 0)),
        ],
        out_specs=pl.BlockSpec((1, 1, 1024), lambda b, t: (b, 0, 0)),
        out_shape=jax.ShapeDtypeStruct((B, 1, 1024), f32),
    )(m, W3, b3r)

    bf1r = bf1.reshape(1, -1)
    bf2r = bf2.reshape(1, -1)
    bf3r = bf3.reshape(1, -1)
    out = pl.pallas_call(
        functools.partial(_stage_d, c_dim=C),
        grid=(B,),
        in_specs=[
            pl.BlockSpec((1, 1, 1024), lambda b: (b, 0, 0)),
            pl.BlockSpec(Wf1.shape, lambda b: (0, 0)),
            pl.BlockSpec(bf1r.shape, lambda b: (0, 0)),
            pl.BlockSpec(Wf2.shape, lambda b: (0, 0)),
            pl.BlockSpec(bf2r.shape, lambda b: (0, 0)),
            pl.BlockSpec(Wf3.shape, lambda b: (0, 0)),
            pl.BlockSpec(bf3r.shape, lambda b: (0, 0)),
            pl.BlockSpec((1, C, N), lambda b: (b, 0, 0)),
        ],
        out_specs=pl.BlockSpec((1, C, N), lambda b: (b, 0, 0)),
        out_shape=jax.ShapeDtypeStruct((B, C, N), f32),
    )(g, Wf1, bf1r, Wf2, bf2r, Wf3, bf3r, x)
    return out
